# Initial kernel scaffold; baseline (speedup 1.0000x reference)
import functools

import jax
import jax.numpy as jnp
from jax.experimental import pallas as pl
from jax.experimental.pallas import tpu as pltpu

B, S, H = 1, 2048, 1024
E, I, K = 8, 512, 2
NG, TG = 4, 2
RSF = 2.5
TT = 256
BIG = jnp.float32(-1e30)


def _routing_wfull(x, rw, eb):
    logits = jax.lax.dot_general(
        x, rw, (((1,), (1,)), ((), ())), preferred_element_type=jnp.float32)
    scores = jax.nn.sigmoid(logits)
    sfc = scores + eb
    lane = jax.lax.broadcasted_iota(jnp.int32, (TT, E), 1)
    grp = lane // (E // NG)
    li = jax.lax.broadcasted_iota(jnp.int32, (E, E), 0)
    lj = jax.lax.broadcasted_iota(jnp.int32, (E, E), 1)
    pairm = (li // (E // NG) == lj // (E // NG)).astype(jnp.float32)
    gsum = jax.lax.dot_general(
        sfc, pairm, (((1,), (0,)), ((), ())), preferred_element_type=jnp.float32)
    m1 = jnp.max(gsum, axis=1, keepdims=True)
    l1 = jnp.min(jnp.where(gsum == m1, lane, E), axis=1, keepdims=True)
    g1 = l1 // (E // NG)
    rest = grp != g1
    m2 = jnp.max(jnp.where(rest, gsum, BIG), axis=1, keepdims=True)
    l2 = jnp.min(jnp.where(rest & (gsum == m2), lane, E), axis=1, keepdims=True)
    g2 = l2 // (E // NG)
    group_mask = (grp == g1) | (grp == g2)
    cand = jnp.where(group_mask, sfc, 0.0)
    c1 = jnp.max(cand, axis=1, keepdims=True)
    i0 = jnp.min(jnp.where(cand == c1, lane, E), axis=1, keepdims=True)
    reste = lane != i0
    c2 = jnp.max(jnp.where(reste, cand, BIG), axis=1, keepdims=True)
    i1 = jnp.min(jnp.where(reste & (cand == c2), lane, E), axis=1, keepdims=True)
    w0 = jnp.sum(jnp.where(lane == i0, scores, 0.0), axis=1, keepdims=True)
    w1 = jnp.sum(jnp.where(lane == i1, scores, 0.0), axis=1, keepdims=True)
    scale = RSF / (w0 + w1 + 1e-20)
    w_full = (jnp.where(lane == i0, w0, 0.0) + jnp.where(lane == i1, w1, 0.0)) * scale
    return w_full


def _silu(v):
    return v * jax.nn.sigmoid(v)


def _moe_body(x_ref, rw_ref, eb_ref, gp_ref, up_ref, dp_ref,
              sgw_ref, suw_ref, sdw_ref, out_ref):
    x = x_ref[...]
    w_full = _routing_wfull(x, rw_ref[...], eb_ref[...])
    sg = jax.lax.dot_general(x, sgw_ref[...], (((1,), (1,)), ((), ())),
                             preferred_element_type=jnp.float32)
    su = jax.lax.dot_general(x, suw_ref[...], (((1,), (1,)), ((), ())),
                             preferred_element_type=jnp.float32)
    acc = jax.lax.dot_general(_silu(sg) * su, sdw_ref[...],
                              (((1,), (1,)), ((), ())),
                              preferred_element_type=jnp.float32)
    for e in range(E):
        hg = jax.lax.dot_general(x, gp_ref[e], (((1,), (1,)), ((), ())),
                                 preferred_element_type=jnp.float32)
        hu = jax.lax.dot_general(x, up_ref[e], (((1,), (1,)), ((), ())),
                                 preferred_element_type=jnp.float32)
        act = _silu(hg) * hu
        ye = jax.lax.dot_general(act, dp_ref[e], (((1,), (1,)), ((), ())),
                                 preferred_element_type=jnp.float32)
        acc = acc + w_full[:, e:e + 1] * ye
    out_ref[...] = acc


@jax.jit
def _moe(x2d, rw, eb2, gp, up, dp, sgw, suw, sdw):
    T = x2d.shape[0]
    grid = (T // TT,)
    full = lambda shape: pl.BlockSpec(shape, lambda i: tuple(0 for _ in shape))
    return pl.pallas_call(
        _moe_body,
        grid=grid,
        in_specs=[
            pl.BlockSpec((TT, H), lambda i: (i, 0)),
            full((E, H)),
            full((1, E)),
            full((E, I, H)),
            full((E, I, H)),
            full((E, H, I)),
            full((I, H)),
            full((I, H)),
            full((H, I)),
        ],
        out_specs=pl.BlockSpec((TT, H), lambda i: (i, 0)),
        out_shape=jax.ShapeDtypeStruct((T, H), jnp.float32),
    )(x2d, rw, eb2, gp, up, dp, sgw, suw, sdw)


def kernel(hidden_states, router_weight, e_bias, gate_proj, up_proj, down_proj,
           shared_gate_w, shared_up_w, shared_down_w):
    bsz, seq, h = hidden_states.shape
    x2d = hidden_states.reshape(bsz * seq, h)
    eb2 = e_bias.reshape(1, E)
    y = _moe(x2d, router_weight, eb2, gate_proj, up_proj, down_proj,
             shared_gate_w, shared_up_w, shared_down_w)
    return y.reshape(bsz, seq, h)


# speedup vs baseline: 1.9326x; 1.9326x over previous
import functools

import jax
import jax.numpy as jnp
from jax.experimental import pallas as pl
from jax.experimental.pallas import tpu as pltpu

B, S, H = 1, 2048, 1024
E, I, K = 8, 512, 2
NG, TG = 4, 2
RSF = 2.5
TT = 256
BIG = -1e30


def _routing_wfull(x, rw, eb):
    logits = jax.lax.dot_general(
        x, rw, (((1,), (1,)), ((), ())), preferred_element_type=jnp.float32)
    scores = jax.nn.sigmoid(logits)
    sfc = scores + eb
    lane = jax.lax.broadcasted_iota(jnp.int32, (TT, E), 1)
    grp = lane // (E // NG)
    li = jax.lax.broadcasted_iota(jnp.int32, (E, E), 0)
    lj = jax.lax.broadcasted_iota(jnp.int32, (E, E), 1)
    pairm = (li // (E // NG) == lj // (E // NG)).astype(jnp.float32)
    gsum = jax.lax.dot_general(
        sfc, pairm, (((1,), (0,)), ((), ())), preferred_element_type=jnp.float32)
    m1 = jnp.max(gsum, axis=1, keepdims=True)
    l1 = jnp.min(jnp.where(gsum == m1, lane, E), axis=1, keepdims=True)
    g1 = l1 // (E // NG)
    rest = grp != g1
    m2 = jnp.max(jnp.where(rest, gsum, BIG), axis=1, keepdims=True)
    l2 = jnp.min(jnp.where(rest & (gsum == m2), lane, E), axis=1, keepdims=True)
    g2 = l2 // (E // NG)
    group_mask = (grp == g1) | (grp == g2)
    cand = jnp.where(group_mask, sfc, 0.0)
    c1 = jnp.max(cand, axis=1, keepdims=True)
    i0 = jnp.min(jnp.where(cand == c1, lane, E), axis=1, keepdims=True)
    reste = lane != i0
    c2 = jnp.max(jnp.where(reste, cand, BIG), axis=1, keepdims=True)
    i1 = jnp.min(jnp.where(reste & (cand == c2), lane, E), axis=1, keepdims=True)
    w0 = jnp.sum(jnp.where(lane == i0, scores, 0.0), axis=1, keepdims=True)
    w1 = jnp.sum(jnp.where(lane == i1, scores, 0.0), axis=1, keepdims=True)
    scale = RSF / (w0 + w1 + 1e-20)
    w_full = (jnp.where(lane == i0, w0, 0.0) + jnp.where(lane == i1, w1, 0.0)) * scale
    return w_full


def _silu(v):
    return v * jax.nn.sigmoid(v)


def _moe_body(x_ref, rw_ref, eb_ref, gp_ref, up_ref, dp_ref,
              sgw_ref, suw_ref, sdw_ref, out_ref):
    x = x_ref[...]
    w_full = _routing_wfull(x, rw_ref[...], eb_ref[...])
    xb = x.astype(jnp.bfloat16)
    sg = jax.lax.dot_general(xb, sgw_ref[...], (((1,), (1,)), ((), ())),
                             preferred_element_type=jnp.float32)
    su = jax.lax.dot_general(xb, suw_ref[...], (((1,), (1,)), ((), ())),
                             preferred_element_type=jnp.float32)
    acc = jax.lax.dot_general((_silu(sg) * su).astype(jnp.bfloat16),
                              sdw_ref[...], (((1,), (1,)), ((), ())),
                              preferred_element_type=jnp.float32)
    for e in range(E):
        hg = jax.lax.dot_general(xb, gp_ref[e], (((1,), (1,)), ((), ())),
                                 preferred_element_type=jnp.float32)
        hu = jax.lax.dot_general(xb, up_ref[e], (((1,), (1,)), ((), ())),
                                 preferred_element_type=jnp.float32)
        act = (_silu(hg) * hu).astype(jnp.bfloat16)
        ye = jax.lax.dot_general(act, dp_ref[e], (((1,), (1,)), ((), ())),
                                 preferred_element_type=jnp.float32)
        acc = acc + w_full[:, e:e + 1] * ye
    out_ref[...] = acc


@jax.jit
def _moe(x2d, rw, eb2, gp, up, dp, sgw, suw, sdw):
    T = x2d.shape[0]
    grid = (T // TT,)
    full = lambda shape: pl.BlockSpec(shape, lambda i: tuple(0 for _ in shape))
    return pl.pallas_call(
        _moe_body,
        grid=grid,
        in_specs=[
            pl.BlockSpec((TT, H), lambda i: (i, 0)),
            full((E, H)),
            full((1, E)),
            full((E, I, H)),
            full((E, I, H)),
            full((E, H, I)),
            full((I, H)),
            full((I, H)),
            full((H, I)),
        ],
        out_specs=pl.BlockSpec((TT, H), lambda i: (i, 0)),
        out_shape=jax.ShapeDtypeStruct((T, H), jnp.float32),
    )(x2d, rw, eb2, gp, up, dp, sgw, suw, sdw)


def kernel(hidden_states, router_weight, e_bias, gate_proj, up_proj, down_proj,
           shared_gate_w, shared_up_w, shared_down_w):
    bsz, seq, h = hidden_states.shape
    x2d = hidden_states.reshape(bsz * seq, h)
    eb2 = e_bias.reshape(1, E)
    bf = jnp.bfloat16
    y = _moe(x2d, router_weight, eb2, gate_proj.astype(bf), up_proj.astype(bf),
             down_proj.astype(bf), shared_gate_w.astype(bf),
             shared_up_w.astype(bf), shared_down_w.astype(bf))
    return y.reshape(bsz, seq, h)
